# Initial kernel scaffold; baseline (speedup 1.0000x reference)
import jax
import jax.numpy as jnp
from jax.experimental import pallas as pl


def kernel(x, gate_w, gate_b, expert_w, expert_b):
    raise NotImplementedError("write your pallas kernel here")


# speedup vs baseline: 1.7212x; 1.7212x over previous
import functools

import jax
import jax.numpy as jnp
from jax.experimental import pallas as pl
from jax.experimental.pallas import tpu as pltpu

_EPS = 1e-08
_ENTROPY_WEIGHT = 0.05
_MAX_USAGE_RATIO = 0.4

_T_ROUTER = 512
_TM_DENSE = 512


def _router_body(x_ref, gw_ref, gb_ref, probs_ref, idx_ref, rank_ref,
                 counts_ref, ent_ref):
    i = pl.program_id(0)
    T = x_ref.shape[0]
    E = gw_ref.shape[0]

    @pl.when(i == 0)
    def _():
        counts_ref[...] = jnp.zeros_like(counts_ref)
        ent_ref[...] = jnp.zeros_like(ent_ref)

    x = x_ref[...]
    logits = jax.lax.dot_general(
        x, gw_ref[...], (((1,), (1,)), ((), ())),
        preferred_element_type=jnp.float32,
        precision=jax.lax.Precision.DEFAULT)
    logits = logits + gb_ref[...]
    m = jnp.max(logits, axis=1, keepdims=True)
    ex = jnp.exp(logits - m)
    probs = ex / jnp.sum(ex, axis=1, keepdims=True)
    ent_tile = -jnp.sum(probs * jnp.log(probs + _EPS), axis=(0, 1),
                        keepdims=True)

    cols = jax.lax.broadcasted_iota(jnp.int32, (T, E), 1)
    m1 = jnp.max(probs, axis=1, keepdims=True)
    i1 = jnp.min(jnp.where(probs >= m1, cols, E), axis=1, keepdims=True)
    h1 = cols == i1
    probsm = jnp.where(h1, -jnp.inf, probs)
    m2 = jnp.max(probsm, axis=1, keepdims=True)
    i2 = jnp.min(jnp.where(probsm >= m2, cols, E), axis=1, keepdims=True)
    h2 = cols == i2

    h1f = h1.astype(jnp.float32)
    h2f = h2.astype(jnp.float32)
    hh = h1f + h2f
    r_i = jax.lax.broadcasted_iota(jnp.int32, (T, T), 0)
    c_i = jax.lax.broadcasted_iota(jnp.int32, (T, T), 1)
    tri = (r_i > c_i).astype(jnp.float32)
    c0 = jax.lax.dot_general(
        tri, hh, (((1,), (0,)), ((), ())),
        preferred_element_type=jnp.float32,
        precision=jax.lax.Precision.HIGHEST)
    base = counts_ref[...] + c0
    r1 = jnp.sum(base * h1f, axis=1)
    r2 = jnp.sum(base * h2f, axis=1)

    probs_ref[0, 0, :] = m1[:, 0]
    probs_ref[0, 1, :] = m2[:, 0]
    idx_ref[0, 0, :] = i1[:, 0]
    idx_ref[0, 1, :] = i2[:, 0]
    rank_ref[0, 0, :] = r1.astype(jnp.int32)
    rank_ref[0, 1, :] = r2.astype(jnp.int32)
    counts_ref[...] = counts_ref[...] + jnp.sum(hh, axis=0, keepdims=True)
    ent_ref[...] = ent_ref[...] + ent_tile


def _run_router(x_flat, gate_w, gate_b2d, interpret=False):
    n, d = x_flat.shape
    e = gate_w.shape[0]
    nt = n // _T_ROUTER
    out_shape = [
        jax.ShapeDtypeStruct((nt, 2, _T_ROUTER), jnp.float32),
        jax.ShapeDtypeStruct((nt, 2, _T_ROUTER), jnp.int32),
        jax.ShapeDtypeStruct((nt, 2, _T_ROUTER), jnp.int32),
        jax.ShapeDtypeStruct((1, e), jnp.float32),
        jax.ShapeDtypeStruct((1, 1), jnp.float32),
    ]
    in_specs = [
        pl.BlockSpec((_T_ROUTER, d), lambda i: (i, 0)),
        pl.BlockSpec((e, d), lambda i: (0, 0)),
        pl.BlockSpec((1, e), lambda i: (0, 0)),
    ]
    tile3 = pl.BlockSpec((1, 2, _T_ROUTER), lambda i: (i, 0, 0))
    out_specs = [
        tile3, tile3, tile3,
        pl.BlockSpec((1, e), lambda i: (0, 0)),
        pl.BlockSpec((1, 1), lambda i: (0, 0)),
    ]
    return pl.pallas_call(
        _router_body, grid=(nt,), in_specs=in_specs, out_specs=out_specs,
        out_shape=out_shape, interpret=interpret,
    )(x_flat, gate_w, gate_b2d)


def _dense_body(x_ref, w_ref, b_ref, probs_ref, idx_ref, out_ref):
    e = pl.program_id(1)

    @pl.when(e == 0)
    def _():
        out_ref[...] = jnp.zeros_like(out_ref)

    x = x_ref[...].astype(jnp.bfloat16)
    w = w_ref[0].astype(jnp.bfloat16)
    y = jax.lax.dot_general(
        x, w, (((1,), (1,)), ((), ())),
        preferred_element_type=jnp.float32)
    y = y + b_ref[0]
    p1 = probs_ref[0, 0, :]
    p2 = probs_ref[0, 1, :]
    i1 = idx_ref[0, 0, :]
    i2 = idx_ref[0, 1, :]
    scale = (p1 * (i1 == e).astype(jnp.float32)
             + p2 * (i2 == e).astype(jnp.float32))
    out_ref[...] = out_ref[...] + scale[:, None] * y


def _dense_moe(x_flat, expert_w, expert_b, probs, idx, interpret=False):
    n, d = x_flat.shape
    e_num, h, _ = expert_w.shape
    nt = n // _TM_DENSE
    assert _TM_DENSE == _T_ROUTER
    in_specs = [
        pl.BlockSpec((_TM_DENSE, d), lambda i, e: (i, 0)),
        pl.BlockSpec((1, h, d), lambda i, e: (e, 0, 0)),
        pl.BlockSpec((1, 1, h), lambda i, e: (e, 0, 0)),
        pl.BlockSpec((1, 2, _T_ROUTER), lambda i, e: (i, 0, 0)),
        pl.BlockSpec((1, 2, _T_ROUTER), lambda i, e: (i, 0, 0)),
    ]
    out_specs = pl.BlockSpec((_TM_DENSE, h), lambda i, e: (i, 0))
    return pl.pallas_call(
        _dense_body, grid=(nt, e_num), in_specs=in_specs,
        out_specs=out_specs,
        out_shape=jax.ShapeDtypeStruct((n, h), jnp.float32),
        interpret=interpret,
    )(x_flat, expert_w, expert_b.reshape(e_num, 1, h), probs, idx)


def kernel(x, gate_w, gate_b, expert_w, expert_b, interpret=False):
    b, s, d = x.shape
    n = b * s
    x_flat = x.reshape(n, d)
    probs, idx, rank, counts, ent = _run_router(
        x_flat, gate_w, gate_b.reshape(1, -1), interpret=interpret)
    out = _dense_moe(x_flat, expert_w, expert_b, probs, idx,
                     interpret=interpret)
    ent_loss = _ENTROPY_WEIGHT * (ent[0, 0] / n)
    ratios = counts[0] / (n + _EPS)
    loss = ent_loss + jnp.sum(jax.nn.relu(ratios - _MAX_USAGE_RATIO))
    return out.reshape(b, s, -1), loss


# speedup vs baseline: 2.1712x; 1.2614x over previous
import functools

import jax
import jax.numpy as jnp
from jax.experimental import pallas as pl
from jax.experimental.pallas import tpu as pltpu

_EPS = 1e-08
_ENTROPY_WEIGHT = 0.05
_MAX_USAGE_RATIO = 0.4

_T_ROUTER = 1024
_TM_DENSE = 1024


def _router_body(x_ref, gw_ref, gb_ref, probs_ref, idx_ref, rank_ref,
                 counts_ref, ent_ref):
    i = pl.program_id(0)
    T = x_ref.shape[0]
    E = gw_ref.shape[0]

    @pl.when(i == 0)
    def _():
        counts_ref[...] = jnp.zeros_like(counts_ref)
        ent_ref[...] = jnp.zeros_like(ent_ref)

    x = x_ref[...]
    logits = jax.lax.dot_general(
        x, gw_ref[...], (((1,), (1,)), ((), ())),
        preferred_element_type=jnp.float32,
        precision=jax.lax.Precision.DEFAULT)
    logits = logits + gb_ref[...]
    m = jnp.max(logits, axis=1, keepdims=True)
    ex = jnp.exp(logits - m)
    probs = ex / jnp.sum(ex, axis=1, keepdims=True)
    ent_tile = -jnp.sum(probs * jnp.log(probs + _EPS), axis=(0, 1),
                        keepdims=True)

    cols = jax.lax.broadcasted_iota(jnp.int32, (T, E), 1)
    m1 = jnp.max(probs, axis=1, keepdims=True)
    i1 = jnp.min(jnp.where(probs >= m1, cols, E), axis=1, keepdims=True)
    h1 = cols == i1
    probsm = jnp.where(h1, -jnp.inf, probs)
    m2 = jnp.max(probsm, axis=1, keepdims=True)
    i2 = jnp.min(jnp.where(probsm >= m2, cols, E), axis=1, keepdims=True)
    h2 = cols == i2

    h1f = h1.astype(jnp.float32)
    h2f = h2.astype(jnp.float32)
    hh = h1f + h2f
    r_i = jax.lax.broadcasted_iota(jnp.int32, (T, T), 0)
    c_i = jax.lax.broadcasted_iota(jnp.int32, (T, T), 1)
    tri = (r_i > c_i).astype(jnp.float32)
    c0 = jax.lax.dot_general(
        tri, hh, (((1,), (0,)), ((), ())),
        preferred_element_type=jnp.float32,
        precision=jax.lax.Precision.HIGHEST)
    base = counts_ref[...] + c0
    r1 = jnp.sum(base * h1f, axis=1)
    r2 = jnp.sum(base * h2f, axis=1)

    probs_ref[0, 0, :] = m1[:, 0]
    probs_ref[0, 1, :] = m2[:, 0]
    idx_ref[0, 0, :] = i1[:, 0]
    idx_ref[0, 1, :] = i2[:, 0]
    rank_ref[0, 0, :] = r1.astype(jnp.int32)
    rank_ref[0, 1, :] = r2.astype(jnp.int32)
    counts_ref[...] = counts_ref[...] + jnp.sum(hh, axis=0, keepdims=True)
    ent_ref[...] = ent_ref[...] + ent_tile


def _run_router(x_flat, gate_w, gate_b2d, interpret=False):
    n, d = x_flat.shape
    e = gate_w.shape[0]
    nt = n // _T_ROUTER
    out_shape = [
        jax.ShapeDtypeStruct((nt, 2, _T_ROUTER), jnp.float32),
        jax.ShapeDtypeStruct((nt, 2, _T_ROUTER), jnp.int32),
        jax.ShapeDtypeStruct((nt, 2, _T_ROUTER), jnp.int32),
        jax.ShapeDtypeStruct((1, e), jnp.float32),
        jax.ShapeDtypeStruct((1, 1), jnp.float32),
    ]
    in_specs = [
        pl.BlockSpec((_T_ROUTER, d), lambda i: (i, 0)),
        pl.BlockSpec((e, d), lambda i: (0, 0)),
        pl.BlockSpec((1, e), lambda i: (0, 0)),
    ]
    tile3 = pl.BlockSpec((1, 2, _T_ROUTER), lambda i: (i, 0, 0))
    out_specs = [
        tile3, tile3, tile3,
        pl.BlockSpec((1, e), lambda i: (0, 0)),
        pl.BlockSpec((1, 1), lambda i: (0, 0)),
    ]
    return pl.pallas_call(
        _router_body, grid=(nt,), in_specs=in_specs, out_specs=out_specs,
        out_shape=out_shape, interpret=interpret,
    )(x_flat, gate_w, gate_b2d)


def _dense_body(x_ref, w_ref, b_ref, probs_ref, idx_ref, out_ref):
    e = pl.program_id(1)

    @pl.when(e == 0)
    def _():
        out_ref[...] = jnp.zeros_like(out_ref)

    x = x_ref[...].astype(jnp.bfloat16)
    w = w_ref[0].astype(jnp.bfloat16)
    y = jax.lax.dot_general(
        x, w, (((1,), (1,)), ((), ())),
        preferred_element_type=jnp.float32)
    y = y + b_ref[0]
    p1 = probs_ref[0, 0, :]
    p2 = probs_ref[0, 1, :]
    i1 = idx_ref[0, 0, :]
    i2 = idx_ref[0, 1, :]
    scale = (p1 * (i1 == e).astype(jnp.float32)
             + p2 * (i2 == e).astype(jnp.float32))
    out_ref[...] = out_ref[...] + scale[:, None] * y


def _dense_moe(x_flat, expert_w, expert_b, probs, idx, interpret=False):
    n, d = x_flat.shape
    e_num, h, _ = expert_w.shape
    nt = n // _TM_DENSE
    assert _TM_DENSE == _T_ROUTER
    in_specs = [
        pl.BlockSpec((_TM_DENSE, d), lambda i, e: (i, 0)),
        pl.BlockSpec((1, h, d), lambda i, e: (e, 0, 0)),
        pl.BlockSpec((1, 1, h), lambda i, e: (e, 0, 0)),
        pl.BlockSpec((1, 2, _T_ROUTER), lambda i, e: (i, 0, 0)),
        pl.BlockSpec((1, 2, _T_ROUTER), lambda i, e: (i, 0, 0)),
    ]
    out_specs = pl.BlockSpec((_TM_DENSE, h), lambda i, e: (i, 0))
    return pl.pallas_call(
        _dense_body, grid=(nt, e_num), in_specs=in_specs,
        out_specs=out_specs,
        out_shape=jax.ShapeDtypeStruct((n, h), jnp.float32),
        interpret=interpret,
    )(x_flat, expert_w, expert_b.reshape(e_num, 1, h), probs, idx)


def kernel(x, gate_w, gate_b, expert_w, expert_b, interpret=False):
    b, s, d = x.shape
    n = b * s
    x_flat = x.reshape(n, d)
    probs, idx, rank, counts, ent = _run_router(
        x_flat, gate_w, gate_b.reshape(1, -1), interpret=interpret)
    out = _dense_moe(x_flat, expert_w, expert_b, probs, idx,
                     interpret=interpret)
    ent_loss = _ENTROPY_WEIGHT * (ent[0, 0] / n)
    ratios = counts[0] / (n + _EPS)
    loss = ent_loss + jnp.sum(jax.nn.relu(ratios - _MAX_USAGE_RATIO))
    return out.reshape(b, s, -1), loss
